# baseline (speedup 1.0000x reference)
import functools

import jax
import jax.numpy as jnp
from jax import lax
from jax.experimental import pallas as pl
from jax.experimental.pallas import tpu as pltpu
from jax.experimental.pallas import tpu_sc as plsc

VOCAB = 50257
D = 256
CTX = 256
BATCH = 1024

NC = 2
NS = 16
NW = NC * NS
NTOK = BATCH * CTX
COLS_PER_W = CTX // NW
BBLK = 64
NBLK = BATCH // BBLK
NCHUNK = COLS_PER_W * NBLK
NBUF = 4
LANES = 16


def _build_kernel():
    mesh = plsc.VectorSubcoreMesh(core_axis_name="c", subcore_axis_name="s")

    @functools.partial(
        pl.kernel,
        mesh=mesh,
        out_type=jax.ShapeDtypeStruct((NTOK, 2, D // 2), jnp.float32),
        scratch_types=[
            pltpu.VMEM((COLS_PER_W, NBLK, BBLK), jnp.int32),
            pltpu.VMEM((COLS_PER_W, NBLK, BBLK), jnp.int32),
            pltpu.VMEM((COLS_PER_W, 2, D // 2), jnp.float32),
        ]
        + [pltpu.VMEM((BBLK, 2, D // 2), jnp.float32) for _ in range(NBUF)]
        + [pltpu.SemaphoreType.DMA for _ in range(2 * NBUF)],
    )
    def emb_kernel(tok_hbm, oidx_hbm, table_hbm, pos_hbm, out_hbm,
                   idx_v, oidx_v, pos_v, *bufs_and_sems):
        bufs = bufs_and_sems[:NBUF]
        gsems = bufs_and_sems[NBUF:2 * NBUF]
        osems = bufs_and_sems[2 * NBUF:]
        wid = lax.axis_index("s") * NC + lax.axis_index("c")
        pltpu.sync_copy(tok_hbm.at[wid], idx_v)
        pltpu.sync_copy(oidx_hbm.at[wid], oidx_v)
        pltpu.sync_copy(pos_hbm.at[pl.ds(wid * COLS_PER_W, COLS_PER_W)], pos_v)

        def kb(c):
            return lax.div(c, NBLK), lax.rem(c, NBLK)

        def g_issue(c, m):
            k, b = kb(c)
            pltpu.async_copy(table_hbm.at[idx_v.at[k, b]], bufs[m], gsems[m])

        def g_wait(c, m):
            k, b = kb(c)
            pltpu.make_async_copy(
                table_hbm.at[idx_v.at[k, b]], bufs[m], gsems[m]).wait()

        def o_issue(c, m):
            k, b = kb(c)
            pltpu.async_copy(bufs[m], out_hbm.at[oidx_v.at[k, b]], osems[m])

        def o_wait(c, m):
            k, b = kb(c)
            pltpu.make_async_copy(
                bufs[m], out_hbm.at[oidx_v.at[k, b]], osems[m]).wait()

        def add(c, m):
            k, _ = kb(c)
            buf = bufs[m]
            pv = [[pos_v[k, h, pl.ds(j * LANES, LANES)]
                   for j in range(D // 2 // LANES)] for h in range(2)]

            @plsc.parallel_loop(0, BBLK, unroll=2)
            def _row(r):
                for h in range(2):
                    for j in range(D // 2 // LANES):
                        plsc.addupdate(
                            buf.at[r, h, pl.ds(j * LANES, LANES)], pv[h][j])

        g_issue(0, 0)
        g_issue(1, 1)

        def quad_body(i, carry):
            base_c = NBUF * i
            for k in range(NBUF):
                c = base_c + k
                m = k
                g_wait(c, m)
                add(c, m)
                o_issue(c, m)
                mp = (k + 2) % NBUF
                if k < 2:
                    @pl.when(i > 0)
                    def _(c=c, mp=mp):
                        o_wait(c - 2, mp)
                    g_issue(c + 2, mp)
                else:
                    o_wait(c - 2, mp)

                    @pl.when(i + 1 < NCHUNK // NBUF)
                    def _(c=c, mp=mp):
                        g_issue(c + 2, mp)
            return carry

        lax.fori_loop(0, NCHUNK // NBUF, quad_body, 0)
        o_wait(NCHUNK - 2, (NCHUNK - 2) % NBUF)
        o_wait(NCHUNK - 1, (NCHUNK - 1) % NBUF)

    return emb_kernel


_EMB = _build_kernel()


def kernel(tokens, token_embedding, positional_encoding):
    tok_cm = tokens.T.reshape(NW, COLS_PER_W, NBLK, BBLK).astype(jnp.int32)
    l_ids = jnp.arange(CTX, dtype=jnp.int32).reshape(CTX, 1)
    b_ids = jnp.arange(BATCH, dtype=jnp.int32).reshape(1, BATCH)
    oidx = (b_ids * CTX + l_ids).reshape(NW, COLS_PER_W, NBLK, BBLK)
    pos3d = positional_encoding.reshape(CTX, 2, D // 2)
    table3d = token_embedding.reshape(VOCAB, 2, D // 2)
    out = _EMB(tok_cm, oidx, table3d, pos3d)
    return out.reshape(BATCH, CTX, D)


# speedup vs baseline: 2.8718x; 2.8718x over previous
import functools

import jax
import jax.numpy as jnp
from jax import lax
from jax.experimental import pallas as pl
from jax.experimental.pallas import tpu as pltpu
from jax.experimental.pallas import tpu_sc as plsc

VOCAB = 50257
D = 256
CTX = 256
BATCH = 1024

NC = 2
NS = 16
NW = NC * NS
NTOK = BATCH * CTX
COLS_PER_W = CTX // NW
BBLK = 128
NBLK = BATCH // BBLK
NCHUNK = COLS_PER_W * NBLK
LANES = 16


def _build_kernel():
    mesh = plsc.VectorSubcoreMesh(core_axis_name="c", subcore_axis_name="s")

    @functools.partial(
        pl.kernel,
        mesh=mesh,
        out_type=jax.ShapeDtypeStruct((NTOK, D), jnp.float32),
        scratch_types=[
            pltpu.VMEM((COLS_PER_W, NBLK, BBLK), jnp.int32),
            pltpu.VMEM((COLS_PER_W, NBLK, BBLK), jnp.int32),
            pltpu.VMEM((COLS_PER_W, D), jnp.float32),
            pltpu.VMEM((BBLK, D), jnp.float32),
            pltpu.VMEM((BBLK, D), jnp.float32),
            pltpu.SemaphoreType.DMA,
            pltpu.SemaphoreType.DMA,
            pltpu.SemaphoreType.DMA,
        ],
    )
    def emb_kernel(tok_hbm, oidx_hbm, table_hbm, pos_hbm, out_hbm,
                   idx_v, oidx_v, pos_v, buf0, buf1, sem0, sem1, sem_out):
        wid = lax.axis_index("s") * NC + lax.axis_index("c")
        pltpu.sync_copy(tok_hbm.at[wid], idx_v)
        pltpu.sync_copy(oidx_hbm.at[wid], oidx_v)
        pltpu.sync_copy(pos_hbm.at[pl.ds(wid * COLS_PER_W, COLS_PER_W)], pos_v)

        def gather(c, buf, sem):
            k = lax.div(c, NBLK)
            b = lax.rem(c, NBLK)
            pltpu.async_copy(table_hbm.at[idx_v.at[k, b]], buf, sem)

        def wait_gather(c, buf, sem):
            k = lax.div(c, NBLK)
            b = lax.rem(c, NBLK)
            pltpu.make_async_copy(table_hbm.at[idx_v.at[k, b]], buf, sem).wait()

        def add_and_store(c, buf):
            k = lax.div(c, NBLK)
            b = lax.rem(c, NBLK)
            pv = [pos_v[k, pl.ds(j * LANES, LANES)] for j in range(D // LANES)]

            @plsc.parallel_loop(0, BBLK, unroll=2)
            def _row(r):
                for j in range(D // LANES):
                    plsc.addupdate(buf.at[r, pl.ds(j * LANES, LANES)], pv[j])

            copy = pltpu.async_copy(buf, out_hbm.at[oidx_v.at[k, b]], sem_out)
            copy.wait()

        gather(0, buf0, sem0)

        def pair_body(i, carry):
            c0 = 2 * i
            c1 = 2 * i + 1
            gather(c1, buf1, sem1)
            wait_gather(c0, buf0, sem0)
            add_and_store(c0, buf0)

            @pl.when(c0 + 2 < NCHUNK)
            def _():
                gather(c0 + 2, buf0, sem0)

            wait_gather(c1, buf1, sem1)
            add_and_store(c1, buf1)
            return carry

        lax.fori_loop(0, NCHUNK // 2, pair_body, 0)

    return emb_kernel


_EMB = _build_kernel()


def kernel(tokens, token_embedding, positional_encoding):
    tok_cm = tokens.T.reshape(NW, COLS_PER_W, NBLK, BBLK).astype(jnp.int32)
    l_ids = jnp.arange(CTX, dtype=jnp.int32).reshape(CTX, 1)
    b_ids = jnp.arange(BATCH, dtype=jnp.int32).reshape(1, BATCH)
    oidx = (b_ids * CTX + l_ids).reshape(NW, COLS_PER_W, NBLK, BBLK)
    pos2d = positional_encoding.reshape(CTX, D)
    out = _EMB(tok_cm, oidx, token_embedding, pos2d)
    return out.reshape(BATCH, CTX, D)
